# baseline (speedup 1.0000x reference)
import functools

import jax
import jax.numpy as jnp
import numpy as np
from jax import lax
from jax.experimental import pallas as pl
from jax.experimental.pallas import tpu as pltpu
from jax.experimental.pallas import tpu_sc as plsc

N_LEVELS = 16
F_PER_LEVEL = 2
LOG2_T = 19
T = 2 ** LOG2_T
BASE_RES = 16
FINEST_RES = 512
DIM = 3
N_PTS = 524288
MASK = T - 1

_B = (FINEST_RES / BASE_RES) ** (1.0 / (N_LEVELS - 1))
RES = [int(np.floor(BASE_RES * (_B ** lvl))) for lvl in range(N_LEVELS)]

P1 = int(np.uint32(2654435761).view(np.int32))
P2 = int(np.uint32(805459861).view(np.int32))

NW = 32
PTS_PER_W = N_PTS // NW
BATCH = 64
NB = PTS_PER_W // BATCH
NCHUNK = N_LEVELS * 8
OUT_F = N_LEVELS * F_PER_LEVEL


def _body(x_hbm, tab_hbm, mn_hbm, mx_hbm, out_hbm,
          mn_v, mx_v, x_v, xn_t, idx_v, rows_v, out_v, sem):
  nc = 2
  wid = lax.axis_index("s") * nc + lax.axis_index("c")
  pltpu.sync_copy(mn_hbm, mn_v.at[pl.ds(0, DIM)])
  pltpu.sync_copy(mx_hbm, mx_v.at[pl.ds(0, DIM)])
  iota = lax.iota(jnp.int32, 16)
  zero16 = jnp.zeros((16,), jnp.float32)

  vmn = mn_v[...]
  vmx = mx_v[...]
  mn = [jnp.broadcast_to(vmn[d], (16,)) for d in range(DIM)]
  inv = [1.0 / jnp.broadcast_to(vmx[d] - vmn[d], (16,)) for d in range(DIM)]

  def batch_body(t, carry):
    base = wid * PTS_PER_W + t * BATCH
    pltpu.sync_copy(x_hbm.at[pl.ds(base, BATCH)], x_v)

    def g_idx(g, c):
      gb = g * 16
      xs = []
      for d in range(DIM):
        xd = plsc.load_gather(x_v, [gb + iota, jnp.full((16,), d, jnp.int32)])
        xn = (xd - mn[d]) * inv[d]
        xn_t[d, pl.ds(gb, 16)] = xn
        xs.append(xn)
      for l in range(N_LEVELS):
        res = float(RES[l])
        p0 = [(xs[d] * res).astype(jnp.int32) for d in range(DIM)]
        c0a = p0[0]
        c0b = p0[0] + 1
        h1a = p0[1] * P1
        h1b = (p0[1] + 1) * P1
        h2a = p0[2] * P2
        h2b = (p0[2] + 1) * P2
        for corner in range(8):
          b0 = corner & 1
          b1 = (corner >> 1) & 1
          b2 = (corner >> 2) & 1
          h = (c0b if b0 else c0a) ^ (h1b if b1 else h1a) ^ (h2b if b2 else h2a)
          row = (h & MASK) + l * T
          idx_v[l * 8 + corner, pl.ds(gb, 16)] = row
      return c

    lax.fori_loop(0, BATCH // 16, g_idx, 0)

    def fire(j, c):
      pltpu.make_async_copy(tab_hbm.at[idx_v.at[j]], rows_v.at[j], sem).start()
      return c

    lax.fori_loop(0, NCHUNK, fire, 0)

    def drain(j, c):
      pltpu.make_async_copy(tab_hbm.at[idx_v.at[j]], rows_v.at[j], sem).wait()
      return c

    lax.fori_loop(0, NCHUNK, drain, 0)

    def g_acc(g, c):
      gb = g * 16
      ridx = gb + iota
      xs = [xn_t[d, pl.ds(gb, 16)] for d in range(DIM)]
      for l in range(N_LEVELS):
        res = float(RES[l])
        pos = [xs[d] * res for d in range(DIM)]
        p0 = [pos[d].astype(jnp.int32) for d in range(DIM)]
        w = [pos[d] - p0[d].astype(jnp.float32) for d in range(DIM)]
        m = [1.0 - w[d] for d in range(DIM)]
        w01 = [m[0] * m[1], w[0] * m[1], m[0] * w[1], w[0] * w[1]]
        acc0 = zero16
        acc1 = zero16
        for corner in range(8):
          b2 = (corner >> 2) & 1
          wgt = w01[corner & 3] * (w[2] if b2 else m[2])
          ch = jnp.full((16,), l * 8 + corner, jnp.int32)
          f0 = plsc.load_gather(rows_v, [ch, ridx, jnp.full((16,), 0, jnp.int32)])
          f1 = plsc.load_gather(rows_v, [ch, ridx, jnp.full((16,), 1, jnp.int32)])
          acc0 = acc0 + wgt * f0
          acc1 = acc1 + wgt * f1
        plsc.store_scatter(out_v, [ridx, jnp.full((16,), 2 * l, jnp.int32)], acc0)
        plsc.store_scatter(out_v, [ridx, jnp.full((16,), 2 * l + 1, jnp.int32)], acc1)
      return c

    lax.fori_loop(0, BATCH // 16, g_acc, 0)

    pltpu.sync_copy(out_v, out_hbm.at[pl.ds(base, BATCH)])
    return carry

  lax.fori_loop(0, NB, batch_body, 0)


@jax.jit
def _encode_sc(x, tab, mesh_min, mesh_max):
  mesh = plsc.VectorSubcoreMesh(core_axis_name="c", subcore_axis_name="s")
  f = pl.kernel(
      _body,
      out_type=jax.ShapeDtypeStruct((N_PTS, OUT_F), jnp.float32),
      mesh=mesh,
      scratch_types=[
          pltpu.VMEM((16,), jnp.float32),
          pltpu.VMEM((16,), jnp.float32),
          pltpu.VMEM((BATCH, DIM), jnp.float32),
          pltpu.VMEM((DIM, BATCH), jnp.float32),
          pltpu.VMEM((NCHUNK, BATCH), jnp.int32),
          pltpu.VMEM((NCHUNK, BATCH, F_PER_LEVEL), jnp.float32),
          pltpu.VMEM((BATCH, OUT_F), jnp.float32),
          pltpu.SemaphoreType.DMA,
      ],
      compiler_params=pltpu.CompilerParams(
          needs_layout_passes=False, use_tc_tiling_on_sc=False),
  )
  return f(x, tab, mesh_min, mesh_max)


def kernel(x, table, mesh_min, mesh_max):
  tab = table.reshape(N_LEVELS * T, F_PER_LEVEL)
  return _encode_sc(x, tab, mesh_min, mesh_max)


# speedup vs baseline: 4.4298x; 4.4298x over previous
import functools

import jax
import jax.numpy as jnp
import numpy as np
from jax import lax
from jax.experimental import pallas as pl
from jax.experimental.pallas import tpu as pltpu
from jax.experimental.pallas import tpu_sc as plsc

N_LEVELS = 16
F_PER_LEVEL = 2
LOG2_T = 19
T = 2 ** LOG2_T
BASE_RES = 16
FINEST_RES = 512
DIM = 3
N_PTS = 524288
MASK = T - 1

_B = (FINEST_RES / BASE_RES) ** (1.0 / (N_LEVELS - 1))
RES = [int(np.floor(BASE_RES * (_B ** lvl))) for lvl in range(N_LEVELS)]

P1 = int(np.uint32(2654435761).view(np.int32))
P2 = int(np.uint32(805459861).view(np.int32))

NW = 32
PTS_PER_W = N_PTS // NW
BATCH = 64
NB = PTS_PER_W // BATCH
NCHUNK = N_LEVELS * 8
OUT_F = N_LEVELS * F_PER_LEVEL

TAB_WORDS = N_LEVELS * T * F_PER_LEVEL
TAB_ROWS = TAB_WORDS // 8
CV_WORDS = TAB_WORDS // NW
CV_CHUNK = 8192
CV_NCH = CV_WORDS // CV_CHUNK

PT_TILES = N_PTS // 128


def _convert_body(src_hbm, dst_hbm, in_v, out_v):
  nc = 2
  wid = lax.axis_index("s") * nc + lax.axis_index("c")
  iota = lax.iota(jnp.int32, 16)
  iota2 = iota * 2

  def chunk_body(ci, carry):
    base = wid * CV_WORDS + ci * CV_CHUNK
    pltpu.sync_copy(src_hbm.at[pl.ds(base, CV_CHUNK)], in_v)
    for b in range(CV_CHUNK // 256):
      for j in range(8):
        va = in_v[pl.ds(b * 256 + j * 16, 16)]
        vb = in_v[pl.ds(b * 256 + 128 + j * 16, 16)]
        off = b * 256 + j * 32
        plsc.store_scatter(out_v, [off + iota2], va)
        plsc.store_scatter(out_v, [off + 1 + iota2], vb)
    pltpu.sync_copy(out_v, dst_hbm.at[pl.ds(base, CV_CHUNK)])
    return carry

  lax.fori_loop(0, CV_NCH, chunk_body, 0)


def _encode_body(x0_hbm, x1_hbm, x2_hbm, tab_hbm, mn_hbm, mx_hbm, out_hbm,
                 mn_v, mx_v, x_v, idx_v, off_v, rows_v, out_v, sem):
  nc = 2
  wid = lax.axis_index("s") * nc + lax.axis_index("c")
  pltpu.sync_copy(mn_hbm, mn_v.at[pl.ds(0, DIM)])
  pltpu.sync_copy(mx_hbm, mx_v.at[pl.ds(0, DIM)])
  iota = lax.iota(jnp.int32, 16)
  zero16 = jnp.zeros((16,), jnp.float32)

  vmn = mn_v[...]
  vmx = mx_v[...]
  mn = [jnp.broadcast_to(vmn[d], (16,)) for d in range(DIM)]
  inv = [1.0 / jnp.broadcast_to(vmx[d] - vmn[d], (16,)) for d in range(DIM)]
  x_hbms = [x0_hbm, x1_hbm, x2_hbm]

  def batch_body(t, carry):
    base = wid * PTS_PER_W + t * BATCH
    for d in range(DIM):
      pltpu.sync_copy(x_hbms[d].at[pl.ds(base, BATCH)], x_v.at[d])

    def g_idx(g, c):
      gb = g * 16
      xs = []
      for d in range(DIM):
        xd = x_v[d, pl.ds(gb, 16)]
        xn = (xd - mn[d]) * inv[d]
        x_v[DIM + d, pl.ds(gb, 16)] = xn
        xs.append(xn)
      for l in range(N_LEVELS):
        res = float(RES[l])
        p0 = [(xs[d] * res).astype(jnp.int32) for d in range(DIM)]
        c0a = p0[0]
        c0b = p0[0] + 1
        h1a = p0[1] * P1
        h1b = (p0[1] + 1) * P1
        h2a = p0[2] * P2
        h2b = (p0[2] + 1) * P2
        for corner in range(8):
          b0 = corner & 1
          b1 = (corner >> 1) & 1
          b2 = (corner >> 2) & 1
          h = (c0b if b0 else c0a) ^ (h1b if b1 else h1a) ^ (h2b if b2 else h2a)
          hm = h & MASK
          idx_v[l * 8 + corner, pl.ds(gb, 16)] = (
              lax.shift_right_logical(hm, 2) + l * (T // 4))
          off_v[l * 8 + corner, pl.ds(gb, 16)] = (hm & 3) * 2
      return c

    lax.fori_loop(0, BATCH // 16, g_idx, 0)

    def fire(j, c):
      pltpu.make_async_copy(tab_hbm.at[idx_v.at[j]], rows_v.at[j], sem).start()
      return c

    lax.fori_loop(0, NCHUNK, fire, 0)

    def drain(j, c):
      pltpu.make_async_copy(tab_hbm.at[idx_v.at[j]], rows_v.at[j], sem).wait()
      return c

    lax.fori_loop(0, NCHUNK, drain, 0)

    def g_acc(g, c):
      gb = g * 16
      ridx = gb + iota
      xs = [x_v[DIM + d, pl.ds(gb, 16)] for d in range(DIM)]
      for l in range(N_LEVELS):
        res = float(RES[l])
        pos = [xs[d] * res for d in range(DIM)]
        p0 = [pos[d].astype(jnp.int32) for d in range(DIM)]
        w = [pos[d] - p0[d].astype(jnp.float32) for d in range(DIM)]
        m = [1.0 - w[d] for d in range(DIM)]
        w01 = [m[0] * m[1], w[0] * m[1], m[0] * w[1], w[0] * w[1]]
        acc0 = zero16
        acc1 = zero16
        for corner in range(8):
          b2 = (corner >> 2) & 1
          wgt = w01[corner & 3] * (w[2] if b2 else m[2])
          ch = jnp.full((16,), l * 8 + corner, jnp.int32)
          oc = off_v[l * 8 + corner, pl.ds(gb, 16)]
          f0 = plsc.load_gather(rows_v, [ch, ridx, oc])
          f1 = plsc.load_gather(rows_v, [ch, ridx, oc + 1])
          acc0 = acc0 + wgt * f0
          acc1 = acc1 + wgt * f1
        out_v[(2 * l) // 8, (2 * l) % 8, pl.ds(gb, 16)] = acc0
        out_v[(2 * l + 1) // 8, (2 * l + 1) % 8, pl.ds(gb, 16)] = acc1
      return c

    lax.fori_loop(0, BATCH // 16, g_acc, 0)

    pt = lax.shift_right_logical(base, 7)
    p0 = pl.multiple_of(base & 127, BATCH)
    for a in range(4):
      pltpu.sync_copy(out_v.at[a], out_hbm.at[a, pt, :, pl.ds(p0, BATCH)])
    return carry

  lax.fori_loop(0, NB, batch_body, 0)


_SC_PARAMS = pltpu.CompilerParams(
    needs_layout_passes=False, use_tc_tiling_on_sc=False)


@jax.jit
def _hash_encode(x, table, mesh_min, mesh_max):
  mesh = plsc.VectorSubcoreMesh(core_axis_name="c", subcore_axis_name="s")

  t_native = (
      table.reshape(N_LEVELS, T // 128, 128, F_PER_LEVEL)
      .transpose(0, 1, 3, 2)
      .reshape(TAB_WORDS)
  )

  convert = pl.kernel(
      _convert_body,
      out_type=jax.ShapeDtypeStruct((TAB_WORDS,), jnp.float32),
      mesh=mesh,
      scratch_types=[
          pltpu.VMEM((CV_CHUNK,), jnp.float32),
          pltpu.VMEM((CV_CHUNK,), jnp.float32),
      ],
      compiler_params=_SC_PARAMS,
  )
  tab_rows = convert(t_native).reshape(TAB_ROWS, 8)

  x0 = x[:, 0]
  x1 = x[:, 1]
  x2 = x[:, 2]

  encode = pl.kernel(
      _encode_body,
      out_type=jax.ShapeDtypeStruct((4, PT_TILES, 8, 128), jnp.float32),
      mesh=mesh,
      scratch_types=[
          pltpu.VMEM((16,), jnp.float32),
          pltpu.VMEM((16,), jnp.float32),
          pltpu.VMEM((2 * DIM, BATCH), jnp.float32),
          pltpu.VMEM((NCHUNK, BATCH), jnp.int32),
          pltpu.VMEM((NCHUNK, BATCH), jnp.int32),
          pltpu.VMEM((NCHUNK, BATCH, 8), jnp.float32),
          pltpu.VMEM((4, 8, BATCH), jnp.float32),
          pltpu.SemaphoreType.DMA,
      ],
      compiler_params=_SC_PARAMS,
  )
  out_t = encode(x0, x1, x2, tab_rows, mesh_min, mesh_max)
  return out_t.transpose(1, 3, 0, 2).reshape(N_PTS, OUT_F)


def kernel(x, table, mesh_min, mesh_max):
  return _hash_encode(x, table, mesh_min, mesh_max)


# speedup vs baseline: 6.1395x; 1.3860x over previous
import functools

import jax
import jax.numpy as jnp
import numpy as np
from jax import lax
from jax.experimental import pallas as pl
from jax.experimental.pallas import tpu as pltpu
from jax.experimental.pallas import tpu_sc as plsc

N_LEVELS = 16
F_PER_LEVEL = 2
LOG2_T = 19
T = 2 ** LOG2_T
BASE_RES = 16
FINEST_RES = 512
DIM = 3
N_PTS = 524288
MASK = T - 1

_B = (FINEST_RES / BASE_RES) ** (1.0 / (N_LEVELS - 1))
RES = [int(np.floor(BASE_RES * (_B ** lvl))) for lvl in range(N_LEVELS)]

P1 = int(np.uint32(2654435761).view(np.int32))
P2 = int(np.uint32(805459861).view(np.int32))

NW = 32
PTS_PER_W = N_PTS // NW
BATCH = 64
NB = PTS_PER_W // BATCH
NPAIR = NB // 2
NG = BATCH // 16
NCHUNK = N_LEVELS * 8 * BATCH // 128
HALF = NCHUNK // 2
OUT_F = N_LEVELS * F_PER_LEVEL

TAB_WORDS = N_LEVELS * T * F_PER_LEVEL
TAB_ROWS = TAB_WORDS // 8
CV_WORDS = TAB_WORDS // NW
CV_CHUNK = 8192
CV_NCH = CV_WORDS // CV_CHUNK

PT_TILES = N_PTS // 128


def _convert_body(src_hbm, dst_hbm, in_v, out_v):
  nc = 2
  wid = lax.axis_index("s") * nc + lax.axis_index("c")
  iota = lax.iota(jnp.int32, 16)
  iota2 = iota * 2

  def chunk_body(ci, carry):
    base = wid * CV_WORDS + ci * CV_CHUNK
    pltpu.sync_copy(src_hbm.at[pl.ds(base, CV_CHUNK)], in_v)
    for b in range(CV_CHUNK // 256):
      for j in range(8):
        va = in_v[pl.ds(b * 256 + j * 16, 16)]
        vb = in_v[pl.ds(b * 256 + 128 + j * 16, 16)]
        off = b * 256 + j * 32
        plsc.store_scatter(out_v, [off + iota2], va)
        plsc.store_scatter(out_v, [off + 1 + iota2], vb)
    pltpu.sync_copy(out_v, dst_hbm.at[pl.ds(base, CV_CHUNK)])
    return carry

  lax.fori_loop(0, CV_NCH, chunk_body, 0)


def _encode_body(x0_hbm, x1_hbm, x2_hbm, tab_hbm, mn_hbm, mx_hbm, out_hbm,
                 mn_v, mx_v, x_v, xn_a, xn_b, idx_a, idx_b, off_a, off_b,
                 rows_v, out_v, sem0, sem1):
  nc = 2
  wid = lax.axis_index("s") * nc + lax.axis_index("c")
  pltpu.sync_copy(mn_hbm, mn_v.at[pl.ds(0, DIM)])
  pltpu.sync_copy(mx_hbm, mx_v.at[pl.ds(0, DIM)])
  iota = lax.iota(jnp.int32, 16)
  zero16 = jnp.zeros((16,), jnp.float32)

  vmn = mn_v[...]
  vmx = mx_v[...]
  mn = [jnp.broadcast_to(vmn[d], (16,)) for d in range(DIM)]
  inv = [1.0 / jnp.broadcast_to(vmx[d] - vmn[d], (16,)) for d in range(DIM)]
  x_hbms = [x0_hbm, x1_hbm, x2_hbm]

  def xload(pair):
    base = wid * PTS_PER_W + pair * (2 * BATCH)
    for d in range(DIM):
      pltpu.sync_copy(x_hbms[d].at[pl.ds(base, 2 * BATCH)], x_v.at[d])

  def phase2(xoff, xn_v, idx_v, off_v):
    def g_idx(g, c):
      gb = g * 16
      xs = []
      for d in range(DIM):
        xd = x_v[d, pl.ds(xoff + gb, 16)]
        xn = (xd - mn[d]) * inv[d]
        xn_v[d, pl.ds(gb, 16)] = xn
        xs.append(xn)
      for l in range(N_LEVELS):
        res = float(RES[l])
        p0 = [(xs[d] * res).astype(jnp.int32) for d in range(DIM)]
        c0a = p0[0]
        c0b = p0[0] + 1
        h1a = p0[1] * P1
        h1b = (p0[1] + 1) * P1
        h2a = p0[2] * P2
        h2b = (p0[2] + 1) * P2
        for corner in range(8):
          b0 = corner & 1
          b1 = (corner >> 1) & 1
          b2 = (corner >> 2) & 1
          h = (c0b if b0 else c0a) ^ (h1b if b1 else h1a) ^ (h2b if b2 else h2a)
          hm = h & MASK
          p_id = l * 8 + corner
          pos = (p_id & 1) * BATCH + gb
          idx_v[p_id // 2, pl.ds(pos, 16)] = (
              lax.shift_right_logical(hm, 2) + l * (T // 4))
          off_v[p_id // 2, pl.ds(pos, 16)] = (hm & 3) * 2
      return c

    lax.fori_loop(0, NG, g_idx, 0)

  def fire(idx_v, lo, sem):
    def go(j, c):
      pltpu.make_async_copy(
          tab_hbm.at[idx_v.at[j]], rows_v.at[j], sem).start()
      return c
    lax.fori_loop(lo, lo + HALF, go, 0)

  def drain(idx_v, lo, sem):
    def go(j, c):
      pltpu.make_async_copy(
          tab_hbm.at[idx_v.at[j]], rows_v.at[j], sem).wait()
      return c
    lax.fori_loop(lo, lo + HALF, go, 0)

  def phase3(l0, xn_v, off_v):
    def g_acc(g, c):
      gb = g * 16
      xs = [xn_v[d, pl.ds(gb, 16)] for d in range(DIM)]
      for l in range(l0, l0 + 8):
        res = float(RES[l])
        pos = [xs[d] * res for d in range(DIM)]
        p0 = [pos[d].astype(jnp.int32) for d in range(DIM)]
        w = [pos[d] - p0[d].astype(jnp.float32) for d in range(DIM)]
        m = [1.0 - w[d] for d in range(DIM)]
        w01 = [m[0] * m[1], w[0] * m[1], m[0] * w[1], w[0] * w[1]]
        acc0 = zero16
        acc1 = zero16
        for corner in range(8):
          b2 = (corner >> 2) & 1
          wgt = w01[corner & 3] * (w[2] if b2 else m[2])
          p_id = l * 8 + corner
          pos_r = (p_id & 1) * BATCH + gb
          ch = jnp.full((16,), p_id // 2, jnp.int32)
          ridx = pos_r + iota
          oc = off_v[p_id // 2, pl.ds(pos_r, 16)]
          f0 = plsc.load_gather(rows_v, [ch, ridx, oc])
          f1 = plsc.load_gather(rows_v, [ch, ridx, oc + 1])
          acc0 = acc0 + wgt * f0
          acc1 = acc1 + wgt * f1
        out_v[(2 * l) // 8, (2 * l) % 8, pl.ds(gb, 16)] = acc0
        out_v[(2 * l + 1) // 8, (2 * l + 1) % 8, pl.ds(gb, 16)] = acc1
      return c

    lax.fori_loop(0, NG, g_acc, 0)

  def outdma(t):
    base = wid * PTS_PER_W + t * BATCH
    pt = lax.shift_right_logical(base, 7)
    p0 = pl.multiple_of(base & 127, BATCH)
    for a in range(4):
      pltpu.sync_copy(out_v.at[a], out_hbm.at[a, pt, :, pl.ds(p0, BATCH)])

  xload(0)
  phase2(0, xn_a, idx_a, off_a)
  fire(idx_a, 0, sem0)
  fire(idx_a, HALF, sem1)

  def pair_body(p, carry):
    a = 2 * p
    b = a + 1
    drain(idx_a, 0, sem0)
    phase3(0, xn_a, off_a)
    phase2(BATCH, xn_b, idx_b, off_b)
    fire(idx_b, 0, sem0)
    drain(idx_a, HALF, sem1)
    phase3(8, xn_a, off_a)
    outdma(a)
    fire(idx_b, HALF, sem1)
    @pl.when(p < NPAIR - 1)
    def _prep_next():
      xload(p + 1)
      phase2(0, xn_a, idx_a, off_a)
    drain(idx_b, 0, sem0)
    phase3(0, xn_b, off_b)
    @pl.when(p < NPAIR - 1)
    def _fire_next_h1():
      fire(idx_a, 0, sem0)
    drain(idx_b, HALF, sem1)
    phase3(8, xn_b, off_b)
    outdma(b)
    @pl.when(p < NPAIR - 1)
    def _fire_next_h2():
      fire(idx_a, HALF, sem1)
    return carry

  lax.fori_loop(0, NPAIR, pair_body, 0)


_SC_PARAMS = pltpu.CompilerParams(
    needs_layout_passes=False, use_tc_tiling_on_sc=False)


@jax.jit
def _hash_encode(x, table, mesh_min, mesh_max):
  mesh = plsc.VectorSubcoreMesh(core_axis_name="c", subcore_axis_name="s")

  t_native = (
      table.reshape(N_LEVELS, T // 128, 128, F_PER_LEVEL)
      .transpose(0, 1, 3, 2)
      .reshape(TAB_WORDS)
  )

  convert = pl.kernel(
      _convert_body,
      out_type=jax.ShapeDtypeStruct((TAB_WORDS,), jnp.float32),
      mesh=mesh,
      scratch_types=[
          pltpu.VMEM((CV_CHUNK,), jnp.float32),
          pltpu.VMEM((CV_CHUNK,), jnp.float32),
      ],
      compiler_params=_SC_PARAMS,
  )
  tab_rows = convert(t_native).reshape(TAB_ROWS, 8)

  x0 = x[:, 0]
  x1 = x[:, 1]
  x2 = x[:, 2]

  encode = pl.kernel(
      _encode_body,
      out_type=jax.ShapeDtypeStruct((4, PT_TILES, 8, 128), jnp.float32),
      mesh=mesh,
      scratch_types=[
          pltpu.VMEM((16,), jnp.float32),
          pltpu.VMEM((16,), jnp.float32),
          pltpu.VMEM((DIM, 2 * BATCH), jnp.float32),
          pltpu.VMEM((DIM, BATCH), jnp.float32),
          pltpu.VMEM((DIM, BATCH), jnp.float32),
          pltpu.VMEM((NCHUNK, 128), jnp.int32),
          pltpu.VMEM((NCHUNK, 128), jnp.int32),
          pltpu.VMEM((NCHUNK, 128), jnp.int32),
          pltpu.VMEM((NCHUNK, 128), jnp.int32),
          pltpu.VMEM((NCHUNK, 128, 8), jnp.float32),
          pltpu.VMEM((4, 8, BATCH), jnp.float32),
          pltpu.SemaphoreType.DMA,
          pltpu.SemaphoreType.DMA,
      ],
      compiler_params=_SC_PARAMS,
  )
  out_t = encode(x0, x1, x2, tab_rows, mesh_min, mesh_max)
  return out_t.transpose(1, 3, 0, 2).reshape(N_PTS, OUT_F)


def kernel(x, table, mesh_min, mesh_max):
  return _hash_encode(x, table, mesh_min, mesh_max)


# speedup vs baseline: 7.1307x; 1.1615x over previous
import functools

import jax
import jax.numpy as jnp
import numpy as np
from jax import lax
from jax.experimental import pallas as pl
from jax.experimental.pallas import tpu as pltpu
from jax.experimental.pallas import tpu_sc as plsc

N_LEVELS = 16
F_PER_LEVEL = 2
LOG2_T = 19
T = 2 ** LOG2_T
BASE_RES = 16
FINEST_RES = 512
DIM = 3
N_PTS = 524288
MASK = T - 1

_B = (FINEST_RES / BASE_RES) ** (1.0 / (N_LEVELS - 1))
RES = [int(np.floor(BASE_RES * (_B ** lvl))) for lvl in range(N_LEVELS)]

P1 = int(np.uint32(2654435761).view(np.int32))
P2 = int(np.uint32(805459861).view(np.int32))

NW = 32
PTS_PER_W = N_PTS // NW
BATCH = 64
NB = PTS_PER_W // BATCH
NPAIR = NB // 2
NG = BATCH // 16
OUT_F = N_LEVELS * F_PER_LEVEL

N_DENSE = 2
DRES = [RES[l] + 1 for l in range(N_DENSE)]
DCELLS = [r * r * r for r in DRES]
DCELLS_PAD = [(c + 127) // 128 * 128 for c in DCELLS]
DBASE = [0, DCELLS_PAD[0]]
DWORDS = 2 * sum(DCELLS_PAD)
N_HBM_LV = N_LEVELS - N_DENSE
NCHUNK = N_HBM_LV * 8 * BATCH // 128
HALF = NCHUNK // 2
H1_LEVELS = list(range(0, 9))
H2_LEVELS = list(range(9, 16))

TAB_WORDS = N_LEVELS * T * F_PER_LEVEL
TAB_ROWS = TAB_WORDS // 8
CV_WORDS = TAB_WORDS // NW
CV_CHUNK = 8192
CV_NCH = CV_WORDS // CV_CHUNK

PT_TILES = N_PTS // 128


def _convert_body(src_hbm, dst_hbm, in_v, out_v):
  nc = 2
  wid = lax.axis_index("s") * nc + lax.axis_index("c")
  iota = lax.iota(jnp.int32, 16)
  iota2 = iota * 2

  def chunk_body(ci, carry):
    base = wid * CV_WORDS + ci * CV_CHUNK
    pltpu.sync_copy(src_hbm.at[pl.ds(base, CV_CHUNK)], in_v)
    for b in range(CV_CHUNK // 256):
      for j in range(8):
        va = in_v[pl.ds(b * 256 + j * 16, 16)]
        vb = in_v[pl.ds(b * 256 + 128 + j * 16, 16)]
        off = b * 256 + j * 32
        plsc.store_scatter(out_v, [off + iota2], va)
        plsc.store_scatter(out_v, [off + 1 + iota2], vb)
    pltpu.sync_copy(out_v, dst_hbm.at[pl.ds(base, CV_CHUNK)])
    return carry

  lax.fori_loop(0, CV_NCH, chunk_body, 0)


def _encode_body(x0_hbm, x1_hbm, x2_hbm, tab_hbm, mn_hbm, mx_hbm, out_hbm,
                 mn_v, mx_v, x_v, xn_a, xn_b, idx_a, idx_b, off_a, off_b,
                 rows_v, out_v, dense_v, sem0, sem1):
  nc = 2
  wid = lax.axis_index("s") * nc + lax.axis_index("c")
  pltpu.sync_copy(mn_hbm, mn_v.at[pl.ds(0, DIM)])
  pltpu.sync_copy(mx_hbm, mx_v.at[pl.ds(0, DIM)])
  iota = lax.iota(jnp.int32, 16)
  zero16 = jnp.zeros((16,), jnp.float32)

  vmn = mn_v[...]
  vmx = mx_v[...]
  mn = [jnp.broadcast_to(vmn[d], (16,)) for d in range(DIM)]
  inv = [1.0 / jnp.broadcast_to(vmx[d] - vmn[d], (16,)) for d in range(DIM)]
  x_hbms = [x0_hbm, x1_hbm, x2_hbm]

  for l in range(N_DENSE):
    r1 = DRES[l]
    r2 = r1 * r1
    nch = DCELLS_PAD[l] // 128

    def cell_chunk(ci, c, l=l, r1=r1, r2=r2):
      for j in range(8):
        cid = ci * 128 + j * 16 + iota
        cz = cid % r1
        cyx = cid // r1
        cy = cyx % r1
        cx = cyx // r1
        h = cx ^ (cy * P1) ^ (cz * P2)
        hm = h & MASK
        idx_a[0, pl.ds(j * 16, 16)] = (
            lax.shift_right_logical(hm, 2) + l * (T // 4))
        off_a[0, pl.ds(j * 16, 16)] = (hm & 3) * 2
      pltpu.make_async_copy(
          tab_hbm.at[idx_a.at[0]], rows_v.at[0], sem0).start()
      pltpu.make_async_copy(
          tab_hbm.at[idx_a.at[0]], rows_v.at[0], sem0).wait()
      for j in range(8):
        rl = j * 16 + iota
        ch0 = jnp.full((16,), 0, jnp.int32)
        oc = off_a[0, pl.ds(j * 16, 16)]
        f0 = plsc.load_gather(rows_v, [ch0, rl, oc])
        f1 = plsc.load_gather(rows_v, [ch0, rl, oc + 1])
        dci = (DBASE[l] + ci * 128 + j * 16) * 2 + iota * 2
        plsc.store_scatter(dense_v, [dci], f0)
        plsc.store_scatter(dense_v, [dci + 1], f1)
      return c

    lax.fori_loop(0, nch, cell_chunk, 0)

  def xload(pair):
    base = wid * PTS_PER_W + pair * (2 * BATCH)
    for d in range(DIM):
      pltpu.sync_copy(x_hbms[d].at[pl.ds(base, 2 * BATCH)], x_v.at[d])

  def phase2(xoff, xn_v, idx_v, off_v):
    def g_idx(g, c):
      gb = g * 16
      xs = []
      for d in range(DIM):
        xd = x_v[d, pl.ds(xoff + gb, 16)]
        xn = (xd - mn[d]) * inv[d]
        xn_v[d, pl.ds(gb, 16)] = xn
        xs.append(xn)
      for l in range(N_DENSE, N_LEVELS):
        res = float(RES[l])
        p0 = [(xs[d] * res).astype(jnp.int32) for d in range(DIM)]
        c0a = p0[0]
        c0b = p0[0] + 1
        h1a = p0[1] * P1
        h1b = (p0[1] + 1) * P1
        h2a = p0[2] * P2
        h2b = (p0[2] + 1) * P2
        for corner in range(8):
          b0 = corner & 1
          b1 = (corner >> 1) & 1
          b2 = (corner >> 2) & 1
          h = (c0b if b0 else c0a) ^ (h1b if b1 else h1a) ^ (h2b if b2 else h2a)
          hm = h & MASK
          p_id = (l - N_DENSE) * 8 + corner
          pos = (p_id & 1) * BATCH + gb
          idx_v[p_id // 2, pl.ds(pos, 16)] = (
              lax.shift_right_logical(hm, 2) + l * (T // 4))
          off_v[p_id // 2, pl.ds(pos, 16)] = (hm & 3) * 2
      return c

    lax.fori_loop(0, NG, g_idx, 0)

  def fire(idx_v, lo, sem):
    def go(j, c):
      pltpu.make_async_copy(
          tab_hbm.at[idx_v.at[j]], rows_v.at[j], sem).start()
      return c
    lax.fori_loop(lo, lo + HALF, go, 0)

  def drain(idx_v, lo, sem):
    def go(j, c):
      pltpu.make_async_copy(
          tab_hbm.at[idx_v.at[j]], rows_v.at[j], sem).wait()
      return c
    lax.fori_loop(lo, lo + HALF, go, 0)

  def phase3(levels, xn_v, off_v):
    def g_acc(g, c):
      gb = g * 16
      xs = [xn_v[d, pl.ds(gb, 16)] for d in range(DIM)]
      for l in levels:
        res = float(RES[l])
        pos = [xs[d] * res for d in range(DIM)]
        p0 = [pos[d].astype(jnp.int32) for d in range(DIM)]
        w = [pos[d] - p0[d].astype(jnp.float32) for d in range(DIM)]
        m = [1.0 - w[d] for d in range(DIM)]
        w01 = [m[0] * m[1], w[0] * m[1], m[0] * w[1], w[0] * w[1]]
        acc0 = zero16
        acc1 = zero16
        if l < N_DENSE:
          r1 = DRES[l]
          r2 = r1 * r1
          cell = (DBASE[l] + (p0[0] * r1 + p0[1]) * r1 + p0[2]) * 2
          for corner in range(8):
            b0 = corner & 1
            b1 = (corner >> 1) & 1
            b2 = (corner >> 2) & 1
            wgt = w01[corner & 3] * (w[2] if b2 else m[2])
            dci = cell + 2 * (b0 * r2 + b1 * r1 + b2)
            f0 = plsc.load_gather(dense_v, [dci])
            f1 = plsc.load_gather(dense_v, [dci + 1])
            acc0 = acc0 + wgt * f0
            acc1 = acc1 + wgt * f1
        else:
          for corner in range(8):
            b2 = (corner >> 2) & 1
            wgt = w01[corner & 3] * (w[2] if b2 else m[2])
            p_id = (l - N_DENSE) * 8 + corner
            pos_r = (p_id & 1) * BATCH + gb
            ch = jnp.full((16,), p_id // 2, jnp.int32)
            ridx = pos_r + iota
            oc = off_v[p_id // 2, pl.ds(pos_r, 16)]
            f0 = plsc.load_gather(rows_v, [ch, ridx, oc])
            f1 = plsc.load_gather(rows_v, [ch, ridx, oc + 1])
            acc0 = acc0 + wgt * f0
            acc1 = acc1 + wgt * f1
        out_v[(2 * l) // 8, (2 * l) % 8, pl.ds(gb, 16)] = acc0
        out_v[(2 * l + 1) // 8, (2 * l + 1) % 8, pl.ds(gb, 16)] = acc1
      return c

    lax.fori_loop(0, NG, g_acc, 0)

  def outdma(t):
    base = wid * PTS_PER_W + t * BATCH
    pt = lax.shift_right_logical(base, 7)
    p0 = pl.multiple_of(base & 127, BATCH)
    for a in range(4):
      pltpu.sync_copy(out_v.at[a], out_hbm.at[a, pt, :, pl.ds(p0, BATCH)])

  xload(0)
  phase2(0, xn_a, idx_a, off_a)
  fire(idx_a, 0, sem0)
  fire(idx_a, HALF, sem1)

  def pair_body(p, carry):
    a = 2 * p
    b = a + 1
    drain(idx_a, 0, sem0)
    phase3(H1_LEVELS, xn_a, off_a)
    phase2(BATCH, xn_b, idx_b, off_b)
    fire(idx_b, 0, sem0)
    drain(idx_a, HALF, sem1)
    phase3(H2_LEVELS, xn_a, off_a)
    outdma(a)
    fire(idx_b, HALF, sem1)
    @pl.when(p < NPAIR - 1)
    def _prep_next():
      xload(p + 1)
      phase2(0, xn_a, idx_a, off_a)
    drain(idx_b, 0, sem0)
    phase3(H1_LEVELS, xn_b, off_b)
    @pl.when(p < NPAIR - 1)
    def _fire_next_h1():
      fire(idx_a, 0, sem0)
    drain(idx_b, HALF, sem1)
    phase3(H2_LEVELS, xn_b, off_b)
    outdma(b)
    @pl.when(p < NPAIR - 1)
    def _fire_next_h2():
      fire(idx_a, HALF, sem1)
    return carry

  lax.fori_loop(0, NPAIR, pair_body, 0)


_SC_PARAMS = pltpu.CompilerParams(
    needs_layout_passes=False, use_tc_tiling_on_sc=False)


@jax.jit
def _hash_encode(x, table, mesh_min, mesh_max):
  mesh = plsc.VectorSubcoreMesh(core_axis_name="c", subcore_axis_name="s")

  t_native = (
      table.reshape(N_LEVELS, T // 128, 128, F_PER_LEVEL)
      .transpose(0, 1, 3, 2)
      .reshape(TAB_WORDS)
  )

  convert = pl.kernel(
      _convert_body,
      out_type=jax.ShapeDtypeStruct((TAB_WORDS,), jnp.float32),
      mesh=mesh,
      scratch_types=[
          pltpu.VMEM((CV_CHUNK,), jnp.float32),
          pltpu.VMEM((CV_CHUNK,), jnp.float32),
      ],
      compiler_params=_SC_PARAMS,
  )
  tab_rows = convert(t_native).reshape(TAB_ROWS, 8)

  x0 = x[:, 0]
  x1 = x[:, 1]
  x2 = x[:, 2]

  encode = pl.kernel(
      _encode_body,
      out_type=jax.ShapeDtypeStruct((4, PT_TILES, 8, 128), jnp.float32),
      mesh=mesh,
      scratch_types=[
          pltpu.VMEM((16,), jnp.float32),
          pltpu.VMEM((16,), jnp.float32),
          pltpu.VMEM((DIM, 2 * BATCH), jnp.float32),
          pltpu.VMEM((DIM, BATCH), jnp.float32),
          pltpu.VMEM((DIM, BATCH), jnp.float32),
          pltpu.VMEM((NCHUNK, 128), jnp.int32),
          pltpu.VMEM((NCHUNK, 128), jnp.int32),
          pltpu.VMEM((NCHUNK, 128), jnp.int32),
          pltpu.VMEM((NCHUNK, 128), jnp.int32),
          pltpu.VMEM((NCHUNK, 128, 8), jnp.float32),
          pltpu.VMEM((4, 8, BATCH), jnp.float32),
          pltpu.VMEM((DWORDS,), jnp.float32),
          pltpu.SemaphoreType.DMA,
          pltpu.SemaphoreType.DMA,
      ],
      compiler_params=_SC_PARAMS,
  )
  out_t = encode(x0, x1, x2, tab_rows, mesh_min, mesh_max)
  return out_t.transpose(1, 3, 0, 2).reshape(N_PTS, OUT_F)


def kernel(x, table, mesh_min, mesh_max):
  return _hash_encode(x, table, mesh_min, mesh_max)
